# baseline (speedup 1.0000x reference)
import functools

import jax
import jax.numpy as jnp
from jax import lax
from jax.experimental import pallas as pl
from jax.experimental.pallas import tpu as pltpu
from jax.experimental.pallas import tpu_sc as plsc


@functools.cache
def _make_gather(hist: int, batch: int, d: int):
    info = plsc.get_sparse_core_info()
    nc, ns, nl = info.num_cores, info.num_subcores, info.num_lanes
    nw = nc * ns
    assert batch % nw == 0
    bw = batch // nw
    assert hist % 2 == 0
    mesh = plsc.VectorSubcoreMesh(core_axis_name="c", subcore_axis_name="s")

    @functools.partial(
        pl.kernel,
        mesh=mesh,
        compiler_params=pltpu.CompilerParams(
            use_tc_tiling_on_sc=False,
            needs_layout_passes=False,
            disable_bounds_checks=True,
        ),
        out_type=jax.ShapeDtypeStruct((hist, d, batch), jnp.float32),
        scratch_types=[
            pltpu.VMEM((hist, bw), jnp.int32),
            pltpu.VMEM((2, bw, d), jnp.float32),
            pltpu.VMEM((2, d, bw), jnp.float32),
            pltpu.SemaphoreType.DMA((2,)),
            pltpu.SemaphoreType.DMA((2,)),
        ],
    )
    def k(table_hbm, xt_hbm, out_hbm, idx_v, g_v, tr_v, gsem, wsem):
        wid = lax.axis_index("s") * nc + lax.axis_index("c")
        b0 = wid * bw
        pltpu.sync_copy(xt_hbm.at[:, pl.ds(b0, bw)], idx_v)

        def gather_copy(t, b):
            return pltpu.make_async_copy(
                table_hbm.at[idx_v.at[t]], g_v.at[b], gsem.at[b]
            )

        def write_copy(t, b):
            return pltpu.make_async_copy(
                tr_v.at[b], out_hbm.at[t, :, pl.ds(b0, bw)], wsem.at[b]
            )

        lanes = lax.iota(jnp.int32, nl)

        def transpose(b):
            def body(i, carry):
                r = lanes + i * nl
                for dd in range(d):
                    c = (lanes + dd) & (d - 1)
                    v = plsc.load_gather(g_v.at[b], [r, c])
                    plsc.store_scatter(tr_v.at[b], [c, r], v)
                return carry

            lax.fori_loop(0, bw // nl, body, 0)

        gather_copy(0, 0).start()

        def outer(o, carry):
            for b in (0, 1):
                t = 2 * o + b

                @pl.when(t < hist - 1)
                def _():
                    gather_copy(t + 1, 1 - b).start()

                gather_copy(t, b).wait()

                @pl.when(t >= 2)
                def _():
                    write_copy(t - 2, b).wait()

                transpose(b)
                write_copy(t, b).start()
            return carry

        lax.fori_loop(0, hist // 2, outer, 0)
        write_copy(hist - 2, 0).wait()
        write_copy(hist - 1, 1).wait()

    return k


def kernel(x, table):
    b, h = x.shape
    v, d = table.shape
    xt = lax.bitcast_convert_type(
        lax.bitcast_convert_type(x, jnp.float32).T, jnp.int32
    )
    o = _make_gather(h, b, d)(table, xt)
    return o.transpose(2, 0, 1)


# speedup vs baseline: 1.0137x; 1.0137x over previous
import functools

import jax
import jax.numpy as jnp
from jax import lax
from jax.experimental import pallas as pl
from jax.experimental.pallas import tpu as pltpu
from jax.experimental.pallas import tpu_sc as plsc


@functools.cache
def _make_gather(hist: int, batch: int, d: int):
    info = plsc.get_sparse_core_info()
    nc, ns, nl = info.num_cores, info.num_subcores, info.num_lanes
    nw = nc * ns
    assert batch % nw == 0
    bw = batch // nw
    assert hist % 2 == 0
    mesh = plsc.VectorSubcoreMesh(core_axis_name="c", subcore_axis_name="s")

    @functools.partial(
        pl.kernel,
        mesh=mesh,
        compiler_params=pltpu.CompilerParams(
            use_tc_tiling_on_sc=False,
            needs_layout_passes=False,
            disable_bounds_checks=True,
        ),
        out_type=jax.ShapeDtypeStruct((hist, d, batch), jnp.float32),
        scratch_types=[
            pltpu.VMEM((hist, bw), jnp.float32),
            pltpu.VMEM((hist, bw), jnp.int32),
            pltpu.VMEM((2, bw, d), jnp.float32),
            pltpu.VMEM((2, d, bw), jnp.float32),
            pltpu.SemaphoreType.DMA((2,)),
            pltpu.SemaphoreType.DMA((2,)),
        ],
    )
    def k(table_hbm, xt_hbm, out_hbm, idxf_v, idx_v, g_v, tr_v, gsem, wsem):
        wid = lax.axis_index("s") * nc + lax.axis_index("c")
        b0 = wid * bw
        pltpu.sync_copy(xt_hbm.at[:, pl.ds(b0, bw)], idxf_v)

        def tobits(t, carry):
            for i0 in range(0, bw, nl):
                idx_v[t, pl.ds(i0, nl)] = plsc.bitcast(
                    idxf_v[t, pl.ds(i0, nl)], jnp.int32
                )
            return carry

        lax.fori_loop(0, hist, tobits, 0)

        def gather_copy(t, b):
            return pltpu.make_async_copy(
                table_hbm.at[idx_v.at[t]], g_v.at[b], gsem.at[b]
            )

        def write_copy(t, b):
            return pltpu.make_async_copy(
                tr_v.at[b], out_hbm.at[t, :, pl.ds(b0, bw)], wsem.at[b]
            )

        lanes = lax.iota(jnp.int32, nl)

        def transpose(b):
            def body(i, carry):
                r = lanes + i * nl
                for dd in range(d):
                    c = (lanes + dd) & (d - 1)
                    v = plsc.load_gather(g_v.at[b], [r, c])
                    plsc.store_scatter(tr_v.at[b], [c, r], v)
                return carry

            lax.fori_loop(0, bw // nl, body, 0)

        gather_copy(0, 0).start()

        def outer(o, carry):
            for b in (0, 1):
                t = 2 * o + b

                @pl.when(t < hist - 1)
                def _():
                    gather_copy(t + 1, 1 - b).start()

                gather_copy(t, b).wait()

                @pl.when(t >= 2)
                def _():
                    write_copy(t - 2, b).wait()

                transpose(b)
                write_copy(t, b).start()
            return carry

        lax.fori_loop(0, hist // 2, outer, 0)
        write_copy(hist - 2, 0).wait()
        write_copy(hist - 1, 1).wait()

    return k


def kernel(x, table):
    b, h = x.shape
    v, d = table.shape
    xtf = lax.bitcast_convert_type(x, jnp.float32).T
    o = _make_gather(h, b, d)(table, xtf)
    return o.transpose(2, 0, 1)
